# baseline (speedup 1.0000x reference)
import functools

import jax
import jax.numpy as jnp
from jax import lax
from jax.experimental import pallas as pl
from jax.experimental.pallas import tpu as pltpu
from jax.experimental.pallas import tpu_sc as plsc

W = 2048
H = 2048
C = 4
N = 2000000

NC = 2
NS = 16
NW = NC * NS
VEC = 16

CHUNK = 2000
NVEC = CHUNK // VEC
NCHUNKS = N // CHUNK
ITERS = (NCHUNKS + NW - 1) // NW


def _search(c_ref, q, n):
  lo = jnp.zeros((VEC,), jnp.int32)
  step = n >> 1
  while step:
    m = lo + step
    v = plsc.load_gather(c_ref, [m - 1])
    lo = jnp.where(v <= q, m, lo)
    step >>= 1
  cmax = plsc.load_gather(c_ref, [jnp.full((VEC,), n - 1, jnp.int32)])
  cnt = jnp.where(cmax <= q, n, lo)
  xl = cnt - 1
  valid = (xl >= 0) & (xl <= n - 2)
  xlc = jnp.clip(xl, 0, n - 2)
  cl = plsc.load_gather(c_ref, [xlc])
  cu = plsc.load_gather(c_ref, [xlc + 1])
  rd = 1.0 / (cu - cl)
  return xlc, (cu - q) * rd, (q - cl) * rd, valid


SEGP = 2048
PSEG = (H * W) // NW // SEGP


def _make_prep():
  mesh = plsc.VectorSubcoreMesh(core_axis_name="c", subcore_axis_name="s")

  @functools.partial(
      pl.kernel,
      out_type=jax.ShapeDtypeStruct((H * W, 8), jnp.float32),
      mesh=mesh,
      compiler_params=pltpu.CompilerParams(
          needs_layout_passes=False, use_tc_tiling_on_sc=False),
      scratch_types=[
          [pltpu.VMEM((SEGP + 8,), jnp.float32) for _ in range(C)],
          pltpu.VMEM((SEGP, 8), jnp.float32),
      ],
  )
  def prep(zl_hbm, rows_hbm, zbufs, obuf):
    wid = lax.axis_index("s") * NC + lax.axis_index("c")
    p_lo = wid * (PSEG * SEGP)
    lanes = lax.iota(jnp.int32, VEC)

    def seg_body(s, _):
      pbase = p_lo + s * SEGP
      for c in range(C):
        src = c * (H * W) + pbase
        if c == C - 1:
          is_edge = pbase == (H * W - SEGP)

          @pl.when(is_edge)
          def _():
            pltpu.sync_copy(zl_hbm.at[pl.ds(src, SEGP)],
                            zbufs[c].at[pl.ds(0, SEGP)])

          @pl.when(jnp.logical_not(is_edge))
          def _():
            pltpu.sync_copy(zl_hbm.at[pl.ds(src, SEGP + 8)], zbufs[c])
        else:
          pltpu.sync_copy(zl_hbm.at[pl.ds(src, SEGP + 8)], zbufs[c])

      @plsc.parallel_loop(0, SEGP // VEC, unroll=8)
      def _(i):
        row = i * VEC + lanes
        for c in range(C):
          v0 = zbufs[c][pl.ds(i * VEC, VEC)]
          v1 = zbufs[c][pl.ds(i * VEC + 1, VEC)]
          plsc.store_scatter(obuf, [row, jnp.full((VEC,), c, jnp.int32)], v0)
          plsc.store_scatter(obuf, [row, jnp.full((VEC,), c + 4, jnp.int32)],
                             v1)

      pltpu.sync_copy(obuf, rows_hbm.at[pl.ds(pbase, SEGP)])
      return 0

    lax.fori_loop(0, PSEG, seg_body, 0)

  return prep


_prep = _make_prep()


def _make_kernel():
  mesh = plsc.VectorSubcoreMesh(core_axis_name="c", subcore_axis_name="s")

  @functools.partial(
      pl.kernel,
      out_type=[jax.ShapeDtypeStruct((N,), jnp.float32) for _ in range(C)],
      mesh=mesh,
      compiler_params=pltpu.CompilerParams(
          needs_layout_passes=False, use_tc_tiling_on_sc=False),
      scratch_types=[
          pltpu.VMEM((W,), jnp.float32),
          pltpu.VMEM((H,), jnp.float32),
          pltpu.VMEM((CHUNK,), jnp.float32),
          pltpu.VMEM((CHUNK,), jnp.float32),
          pltpu.VMEM((CHUNK,), jnp.int32),
          pltpu.VMEM((CHUNK,), jnp.int32),
          pltpu.VMEM((CHUNK,), jnp.float32),
          pltpu.VMEM((CHUNK,), jnp.float32),
          pltpu.VMEM((CHUNK,), jnp.float32),
          pltpu.VMEM((CHUNK,), jnp.float32),
          pltpu.VMEM((CHUNK,), jnp.float32),
          pltpu.VMEM((CHUNK, 8), jnp.float32),
          pltpu.VMEM((CHUNK, 8), jnp.float32),
          pltpu.VMEM((C, CHUNK), jnp.float32),
          pltpu.SemaphoreType.DMA,
      ],
  )
  def kern(cx_hbm, cy_hbm, xq_hbm, yq_hbm, rows8_hbm,
           o0_hbm, o1_hbm, o2_hbm, o3_hbm,
           cx, cy, xq, yq, idx0, idx1, wxl, wxu, wyl, wyu, msk, g0, g1, ob,
           sem):
    out_hbms = (o0_hbm, o1_hbm, o2_hbm, o3_hbm)
    wid = lax.axis_index("s") * NC + lax.axis_index("c")
    pltpu.sync_copy(cx_hbm, cx)
    pltpu.sync_copy(cy_hbm, cy)

    def chunk_body(i, _):
      cid = i * NW + wid

      @pl.when(cid < NCHUNKS)
      def _():
        base = cid * CHUNK
        pltpu.sync_copy(xq_hbm.at[pl.ds(base, CHUNK)], xq)
        pltpu.sync_copy(yq_hbm.at[pl.ds(base, CHUNK)], yq)

        @plsc.parallel_loop(0, NVEC, unroll=8)
        def _(v):
          off = v * VEC
          qx = xq[pl.ds(off, VEC)]
          qy = yq[pl.ds(off, VEC)]
          xlc, xw_l, xw_u, mx = _search(cx, qx, W)
          ylc, yw_l, yw_u, my = _search(cy, qy, H)
          p0 = ylc * W + xlc
          idx0[pl.ds(off, VEC)] = p0
          idx1[pl.ds(off, VEC)] = p0 + W
          wxl[pl.ds(off, VEC)] = xw_l
          wxu[pl.ds(off, VEC)] = xw_u
          wyl[pl.ds(off, VEC)] = yw_l
          wyu[pl.ds(off, VEC)] = yw_u
          msk[pl.ds(off, VEC)] = jnp.where(mx & my, 1.0, 0.0)

        cp0 = pltpu.async_copy(rows8_hbm.at[idx0], g0, sem)
        cp1 = pltpu.async_copy(rows8_hbm.at[idx1], g1, sem)
        cp0.wait()
        cp1.wait()

        lanes = lax.iota(jnp.int32, VEC)

        @plsc.parallel_loop(0, NVEC, unroll=8)
        def _(v):
          off = v * VEC
          qidx = lanes + off
          axl = wxl[pl.ds(off, VEC)]
          axu = wxu[pl.ds(off, VEC)]
          ayl = wyl[pl.ds(off, VEC)]
          ayu = wyu[pl.ds(off, VEC)]
          m = msk[pl.ds(off, VEC)]
          for c in range(C):
            cf = jnp.full((VEC,), c, jnp.int32)
            cf4 = jnp.full((VEC,), c + 4, jnp.int32)
            r00 = plsc.load_gather(g0, [qidx, cf])
            r01 = plsc.load_gather(g0, [qidx, cf4])
            r10 = plsc.load_gather(g1, [qidx, cf])
            r11 = plsc.load_gather(g1, [qidx, cf4])
            o = ayl * (axl * r00 + axu * r01) + ayu * (axl * r10 + axu * r11)
            o = jnp.where(m != 0.0, o, 0.0)
            ob[c, pl.ds(off, VEC)] = o

        for c in range(C):
          pltpu.sync_copy(ob.at[c], out_hbms[c].at[pl.ds(base, CHUNK)])

      return 0

    lax.fori_loop(0, ITERS, chunk_body, 0)

  return kern


_interp = _make_kernel()

RBLK = 8192
RGRID = -(-N // RBLK)


def _retile_body(i0, i1, i2, i3, o):
  rows = [x[...].reshape(1, RBLK) for x in (i0, i1, i2, i3)]
  o[...] = jnp.concatenate(rows, axis=0)


def _retile(chans):
  return pl.pallas_call(
      _retile_body,
      out_shape=jax.ShapeDtypeStruct((C, N), jnp.float32),
      grid=(RGRID,),
      in_specs=[pl.BlockSpec((RBLK,), lambda j: (j,)) for _ in range(C)],
      out_specs=pl.BlockSpec((C, RBLK), lambda j: (0, j)),
  )(*chans)


@jax.jit
def kernel(x_coords, y_coords, x_query, y_query, z):
  rows8 = _prep(z.reshape(C * H * W))
  chans = _interp(x_coords, y_coords, x_query, y_query, rows8)
  return _retile(chans)


# speedup vs baseline: 1.2284x; 1.2284x over previous
import functools

import jax
import jax.numpy as jnp
from jax import lax
from jax.experimental import pallas as pl
from jax.experimental.pallas import tpu as pltpu
from jax.experimental.pallas import tpu_sc as plsc

W = 2048
H = 2048
C = 4
N = 2000000

NC = 2
NS = 16
NW = NC * NS
VEC = 16

CHUNK = 2000
NVEC = CHUNK // VEC
NCHUNKS = N // CHUNK
ITERS = (NCHUNKS + NW - 1) // NW


def _search(c_ref, q, n):
  lo = jnp.zeros((VEC,), jnp.int32)
  step = n >> 1
  while step:
    m = lo + step
    v = plsc.load_gather(c_ref, [m - 1])
    lo = jnp.where(v <= q, m, lo)
    step >>= 1
  cmax = plsc.load_gather(c_ref, [jnp.full((VEC,), n - 1, jnp.int32)])
  cnt = jnp.where(cmax <= q, n, lo)
  xl = cnt - 1
  valid = (xl >= 0) & (xl <= n - 2)
  xlc = jnp.clip(xl, 0, n - 2)
  cl = plsc.load_gather(c_ref, [xlc])
  cu = plsc.load_gather(c_ref, [xlc + 1])
  rd = 1.0 / (cu - cl)
  return xlc, (cu - q) * rd, (q - cl) * rd, valid


SEGP = 2048
PSEG = (H * W) // NW // SEGP


def _make_prep():
  mesh = plsc.VectorSubcoreMesh(core_axis_name="c", subcore_axis_name="s")

  @functools.partial(
      pl.kernel,
      out_type=jax.ShapeDtypeStruct((H * W, 8), jnp.float32),
      mesh=mesh,
      compiler_params=pltpu.CompilerParams(
          needs_layout_passes=False, use_tc_tiling_on_sc=False),
      scratch_types=[
          [pltpu.VMEM((SEGP + 8,), jnp.float32) for _ in range(C)],
          pltpu.VMEM((SEGP, 8), jnp.float32),
      ],
  )
  def prep(zl_hbm, rows_hbm, zbufs, obuf):
    wid = lax.axis_index("s") * NC + lax.axis_index("c")
    p_lo = wid * (PSEG * SEGP)
    lanes = lax.iota(jnp.int32, VEC)

    def seg_body(s, _):
      pbase = p_lo + s * SEGP
      for c in range(C):
        src = c * (H * W) + pbase
        if c == C - 1:
          is_edge = pbase == (H * W - SEGP)

          @pl.when(is_edge)
          def _():
            pltpu.sync_copy(zl_hbm.at[pl.ds(src, SEGP)],
                            zbufs[c].at[pl.ds(0, SEGP)])

          @pl.when(jnp.logical_not(is_edge))
          def _():
            pltpu.sync_copy(zl_hbm.at[pl.ds(src, SEGP + 8)], zbufs[c])
        else:
          pltpu.sync_copy(zl_hbm.at[pl.ds(src, SEGP + 8)], zbufs[c])

      @plsc.parallel_loop(0, SEGP // VEC, unroll=4)
      def _(i):
        row = i * VEC + lanes
        for c in range(C):
          v0 = zbufs[c][pl.ds(i * VEC, VEC)]
          v1 = zbufs[c][pl.ds(i * VEC + 1, VEC)]
          plsc.store_scatter(obuf, [row, jnp.full((VEC,), c, jnp.int32)], v0)
          plsc.store_scatter(obuf, [row, jnp.full((VEC,), c + 4, jnp.int32)],
                             v1)

      pltpu.sync_copy(obuf, rows_hbm.at[pl.ds(pbase, SEGP)])
      return 0

    lax.fori_loop(0, PSEG, seg_body, 0)

  return prep


_prep = _make_prep()


def _make_kernel():
  mesh = plsc.VectorSubcoreMesh(core_axis_name="c", subcore_axis_name="s")

  @functools.partial(
      pl.kernel,
      out_type=[jax.ShapeDtypeStruct((N,), jnp.float32) for _ in range(C)],
      mesh=mesh,
      compiler_params=pltpu.CompilerParams(
          needs_layout_passes=False, use_tc_tiling_on_sc=False),
      scratch_types=[
          pltpu.VMEM((W,), jnp.float32),
          pltpu.VMEM((H,), jnp.float32),
          [pltpu.VMEM((CHUNK,), jnp.float32) for _ in range(2)],
          [pltpu.VMEM((CHUNK,), jnp.float32) for _ in range(2)],
          [pltpu.VMEM((CHUNK,), jnp.int32) for _ in range(2)],
          [pltpu.VMEM((CHUNK,), jnp.int32) for _ in range(2)],
          [pltpu.VMEM((CHUNK,), jnp.float32) for _ in range(2)],
          [pltpu.VMEM((CHUNK,), jnp.float32) for _ in range(2)],
          [pltpu.VMEM((CHUNK,), jnp.float32) for _ in range(2)],
          [pltpu.VMEM((CHUNK, 8), jnp.float32) for _ in range(2)],
          [pltpu.VMEM((CHUNK, 8), jnp.float32) for _ in range(2)],
          [pltpu.VMEM((C, CHUNK), jnp.float32) for _ in range(2)],
          [pltpu.SemaphoreType.DMA for _ in range(2)],
          [pltpu.SemaphoreType.DMA for _ in range(2)],
          [pltpu.SemaphoreType.DMA for _ in range(2)],
      ],
  )
  def kern(cx_hbm, cy_hbm, xq_hbm, yq_hbm, rows8_hbm,
           o0_hbm, o1_hbm, o2_hbm, o3_hbm,
           cx, cy, xqs, yqs, idx0s, idx1s, wxls, wyls, msks, g0s, g1s, obs,
           semA, semG, semE):
    out_hbms = (o0_hbm, o1_hbm, o2_hbm, o3_hbm)
    wid = lax.axis_index("s") * NC + lax.axis_index("c")
    pltpu.sync_copy(cx_hbm, cx)
    pltpu.sync_copy(cy_hbm, cy)
    lanes = lax.iota(jnp.int32, VEC)

    def valid(i):
      return (jnp.asarray(i, jnp.int32) >= 0) & ((i * NW + wid) < NCHUNKS)

    def qbase(i):
      return (i * NW + wid) * CHUNK

    def issue_a(i, b):
      @pl.when(valid(i))
      def _():
        base = qbase(i)
        pltpu.async_copy(xq_hbm.at[pl.ds(base, CHUNK)], xqs[b], semA[b])
        pltpu.async_copy(yq_hbm.at[pl.ds(base, CHUNK)], yqs[b], semA[b])

    def wait_a(i, b):
      @pl.when(valid(i))
      def _():
        pltpu.make_async_copy(
            xq_hbm.at[pl.ds(0, CHUNK)], xqs[b], semA[b]).wait()
        pltpu.make_async_copy(
            yq_hbm.at[pl.ds(0, CHUNK)], yqs[b], semA[b]).wait()

    def stage_search(i, b):
      @pl.when(valid(i))
      def _():
        xq, yq = xqs[b], yqs[b]
        idx0, idx1 = idx0s[b], idx1s[b]
        wxl, wyl, msk = wxls[b], wyls[b], msks[b]

        @plsc.parallel_loop(0, NVEC, unroll=4)
        def _(v):
          off = v * VEC
          qx = xq[pl.ds(off, VEC)]
          qy = yq[pl.ds(off, VEC)]
          xlc, xw_l, _, mx = _search(cx, qx, W)
          ylc, yw_l, _, my = _search(cy, qy, H)
          p0 = ylc * W + xlc
          idx0[pl.ds(off, VEC)] = p0
          idx1[pl.ds(off, VEC)] = p0 + W
          wxl[pl.ds(off, VEC)] = xw_l
          wyl[pl.ds(off, VEC)] = yw_l
          msk[pl.ds(off, VEC)] = jnp.where(mx & my, 1.0, 0.0)

    def issue_g(i, b):
      @pl.when(valid(i))
      def _():
        pltpu.async_copy(rows8_hbm.at[idx0s[b]], g0s[b], semG[b])
        pltpu.async_copy(rows8_hbm.at[idx1s[b]], g1s[b], semG[b])

    def wait_g(i, b):
      @pl.when(valid(i))
      def _():
        pltpu.make_async_copy(rows8_hbm.at[idx0s[b]], g0s[b], semG[b]).wait()
        pltpu.make_async_copy(rows8_hbm.at[idx1s[b]], g1s[b], semG[b]).wait()

    def stage_reduce(i, b):
      @pl.when(valid(i))
      def _():
        g0, g1, ob = g0s[b], g1s[b], obs[b]
        wxl, wyl, msk = wxls[b], wyls[b], msks[b]

        @plsc.parallel_loop(0, NVEC, unroll=4)
        def _(v):
          off = v * VEC
          qidx = lanes + off
          axl = wxl[pl.ds(off, VEC)]
          ayl = wyl[pl.ds(off, VEC)]
          axu = 1.0 - axl
          ayu = 1.0 - ayl
          m = msk[pl.ds(off, VEC)]
          for c in range(C):
            cf = jnp.full((VEC,), c, jnp.int32)
            cf4 = jnp.full((VEC,), c + 4, jnp.int32)
            r00 = plsc.load_gather(g0, [qidx, cf])
            r01 = plsc.load_gather(g0, [qidx, cf4])
            r10 = plsc.load_gather(g1, [qidx, cf])
            r11 = plsc.load_gather(g1, [qidx, cf4])
            o = ayl * (axl * r00 + axu * r01) + ayu * (axl * r10 + axu * r11)
            o = jnp.where(m != 0.0, o, 0.0)
            ob[c, pl.ds(off, VEC)] = o

    def issue_e(i, b):
      @pl.when(valid(i))
      def _():
        base = qbase(i)
        for c in range(C):
          pltpu.async_copy(obs[b].at[c], out_hbms[c].at[pl.ds(base, CHUNK)],
                           semE[b])

    def wait_e(i, b):
      @pl.when(valid(i))
      def _():
        for c in range(C):
          pltpu.make_async_copy(
              obs[b].at[c], out_hbms[c].at[pl.ds(0, CHUNK)], semE[b]).wait()

    issue_a(0, 0)
    issue_a(1, 1)

    def pipe_body(i2, _):
      for b in range(2):
        i = i2 * 2 + b
        wait_a(i, b)
        stage_search(i, b)
        issue_g(i, b)
        issue_a(i + 2, b)
        j = i - 1
        bj = 1 - b
        wait_e(j - 2, bj)
        wait_g(j, bj)
        stage_reduce(j, bj)
        issue_e(j, bj)
      return 0

    lax.fori_loop(0, ITERS // 2, pipe_body, 0)

    last = ITERS - 1
    wait_e(last - 2, 1)
    wait_g(last, 1)
    stage_reduce(last, 1)
    issue_e(last, 1)
    wait_e(last - 1, 0)
    wait_e(last, 1)

  return kern


_interp = _make_kernel()

RBLK = 8192
RGRID = -(-N // RBLK)


def _retile_body(i0, i1, i2, i3, o):
  rows = [x[...].reshape(1, RBLK) for x in (i0, i1, i2, i3)]
  o[...] = jnp.concatenate(rows, axis=0)


def _retile(chans):
  return pl.pallas_call(
      _retile_body,
      out_shape=jax.ShapeDtypeStruct((C, N), jnp.float32),
      grid=(RGRID,),
      in_specs=[pl.BlockSpec((RBLK,), lambda j: (j,)) for _ in range(C)],
      out_specs=pl.BlockSpec((C, RBLK), lambda j: (0, j)),
  )(*chans)


@jax.jit
def kernel(x_coords, y_coords, x_query, y_query, z):
  rows8 = _prep(z.reshape(C * H * W))
  chans = _interp(x_coords, y_coords, x_query, y_query, rows8)
  return _retile(chans)


# speedup vs baseline: 1.4716x; 1.1980x over previous
import functools

import jax
import jax.numpy as jnp
from jax import lax
from jax.experimental import pallas as pl
from jax.experimental.pallas import tpu as pltpu
from jax.experimental.pallas import tpu_sc as plsc

W = 2048
H = 2048
C = 4
N = 2000000

NC = 2
NS = 16
NW = NC * NS
VEC = 16

CHUNK = 2000
NVEC = CHUNK // VEC
NCHUNKS = N // CHUNK
ITERS = (NCHUNKS + NW - 1) // NW


def _search(c_ref, q, n):
  lo = jnp.zeros((VEC,), jnp.int32)
  step = n >> 1
  while step:
    m = lo + step
    v = plsc.load_gather(c_ref, [m - 1])
    lo = jnp.where(v <= q, m, lo)
    step >>= 1
  cmax = plsc.load_gather(c_ref, [jnp.full((VEC,), n - 1, jnp.int32)])
  cnt = jnp.where(cmax <= q, n, lo)
  xl = cnt - 1
  valid = (xl >= 0) & (xl <= n - 2)
  xlc = jnp.clip(xl, 0, n - 2)
  cl = plsc.load_gather(c_ref, [xlc])
  cu = plsc.load_gather(c_ref, [xlc + 1])
  rd = 1.0 / (cu - cl)
  return xlc, (cu - q) * rd, (q - cl) * rd, valid


SEGP = 2048
PSEG = (H * W) // NW // SEGP


def _make_prep():
  mesh = plsc.VectorSubcoreMesh(core_axis_name="c", subcore_axis_name="s")

  @functools.partial(
      pl.kernel,
      out_type=jax.ShapeDtypeStruct((H * W, 8), jnp.float32),
      mesh=mesh,
      compiler_params=pltpu.CompilerParams(
          needs_layout_passes=False, use_tc_tiling_on_sc=False),
      scratch_types=[
          [[pltpu.VMEM((SEGP + 8,), jnp.float32) for _ in range(C)]
           for _ in range(2)],
          [pltpu.VMEM((SEGP, 8), jnp.float32) for _ in range(2)],
          [pltpu.SemaphoreType.DMA for _ in range(2)],
          [pltpu.SemaphoreType.DMA for _ in range(2)],
      ],
  )
  def prep(zl_hbm, rows_hbm, zbufs, obufs, semI, semO):
    wid = lax.axis_index("s") * NC + lax.axis_index("c")
    p_lo = wid * (PSEG * SEGP)
    lanes = lax.iota(jnp.int32, VEC)

    def issue_i(s, b):
      @pl.when(s < PSEG)
      def _():
        pbase = p_lo + s * SEGP
        for c in range(C):
          src = c * (H * W) + pbase
          if c == C - 1:
            is_edge = pbase == (H * W - SEGP)

            @pl.when(is_edge)
            def _():
              pltpu.async_copy(zl_hbm.at[pl.ds(src, SEGP)],
                               zbufs[b][c].at[pl.ds(0, SEGP)], semI[b])
              pltpu.async_copy(zl_hbm.at[pl.ds(0, 8)],
                               zbufs[b][c].at[pl.ds(SEGP, 8)], semI[b])

            @pl.when(jnp.logical_not(is_edge))
            def _():
              pltpu.async_copy(zl_hbm.at[pl.ds(src, SEGP + 8)], zbufs[b][c],
                               semI[b])
          else:
            pltpu.async_copy(zl_hbm.at[pl.ds(src, SEGP + 8)], zbufs[b][c],
                             semI[b])

    def wait_i(b):
      for c in range(C):
        pltpu.make_async_copy(zl_hbm.at[pl.ds(0, SEGP + 8)], zbufs[b][c],
                              semI[b]).wait()

    def wait_o(s, b):
      @pl.when(s >= 0)
      def _():
        pltpu.make_async_copy(obufs[b], rows_hbm.at[pl.ds(0, SEGP)],
                              semO[b]).wait()

    issue_i(0, 0)
    issue_i(1, 1)

    def seg_body(s2, _):
      for b in range(2):
        s = s2 * 2 + b
        wait_i(b)
        wait_o(s - 2, b)
        obuf = obufs[b]

        @plsc.parallel_loop(0, SEGP // VEC, unroll=4)
        def _(i):
          row = i * VEC + lanes
          for c in range(C):
            v0 = zbufs[b][c][pl.ds(i * VEC, VEC)]
            v1 = zbufs[b][c][pl.ds(i * VEC + 1, VEC)]
            plsc.store_scatter(obuf, [row, jnp.full((VEC,), c, jnp.int32)],
                               v0)
            plsc.store_scatter(obuf, [row, jnp.full((VEC,), c + 4, jnp.int32)],
                               v1)

        pbase = p_lo + s * SEGP
        pltpu.async_copy(obufs[b], rows_hbm.at[pl.ds(pbase, SEGP)], semO[b])
        issue_i(s + 2, b)
      return 0

    lax.fori_loop(0, PSEG // 2, seg_body, 0)
    wait_o(PSEG - 2, 0)
    wait_o(PSEG - 1, 1)

  return prep


_prep = _make_prep()


def _make_kernel():
  mesh = plsc.VectorSubcoreMesh(core_axis_name="c", subcore_axis_name="s")

  @functools.partial(
      pl.kernel,
      out_type=[jax.ShapeDtypeStruct((N,), jnp.float32) for _ in range(C)],
      mesh=mesh,
      compiler_params=pltpu.CompilerParams(
          needs_layout_passes=False, use_tc_tiling_on_sc=False),
      scratch_types=[
          pltpu.VMEM((W,), jnp.float32),
          pltpu.VMEM((H,), jnp.float32),
          [pltpu.VMEM((CHUNK,), jnp.float32) for _ in range(2)],
          [pltpu.VMEM((CHUNK,), jnp.float32) for _ in range(2)],
          [pltpu.VMEM((CHUNK,), jnp.int32) for _ in range(2)],
          [pltpu.VMEM((CHUNK,), jnp.int32) for _ in range(2)],
          [pltpu.VMEM((CHUNK,), jnp.float32) for _ in range(2)],
          [pltpu.VMEM((CHUNK,), jnp.float32) for _ in range(2)],
          [pltpu.VMEM((CHUNK,), jnp.float32) for _ in range(2)],
          [pltpu.VMEM((CHUNK, 8), jnp.float32) for _ in range(2)],
          [pltpu.VMEM((CHUNK, 8), jnp.float32) for _ in range(2)],
          [pltpu.VMEM((C, CHUNK), jnp.float32) for _ in range(2)],
          [pltpu.SemaphoreType.DMA for _ in range(2)],
          [pltpu.SemaphoreType.DMA for _ in range(2)],
          [pltpu.SemaphoreType.DMA for _ in range(2)],
      ],
  )
  def kern(cx_hbm, cy_hbm, xq_hbm, yq_hbm, rows8_hbm,
           o0_hbm, o1_hbm, o2_hbm, o3_hbm,
           cx, cy, xqs, yqs, idx0s, idx1s, wxls, wyls, msks, g0s, g1s, obs,
           semA, semG, semE):
    out_hbms = (o0_hbm, o1_hbm, o2_hbm, o3_hbm)
    wid = lax.axis_index("s") * NC + lax.axis_index("c")
    pltpu.sync_copy(cx_hbm, cx)
    pltpu.sync_copy(cy_hbm, cy)
    lanes = lax.iota(jnp.int32, VEC)

    def valid(i):
      return (jnp.asarray(i, jnp.int32) >= 0) & ((i * NW + wid) < NCHUNKS)

    def qbase(i):
      return (i * NW + wid) * CHUNK

    def issue_a(i, b):
      @pl.when(valid(i))
      def _():
        base = qbase(i)
        pltpu.async_copy(xq_hbm.at[pl.ds(base, CHUNK)], xqs[b], semA[b])
        pltpu.async_copy(yq_hbm.at[pl.ds(base, CHUNK)], yqs[b], semA[b])

    def wait_a(i, b):
      @pl.when(valid(i))
      def _():
        pltpu.make_async_copy(
            xq_hbm.at[pl.ds(0, CHUNK)], xqs[b], semA[b]).wait()
        pltpu.make_async_copy(
            yq_hbm.at[pl.ds(0, CHUNK)], yqs[b], semA[b]).wait()

    def stage_search(i, b):
      @pl.when(valid(i))
      def _():
        xq, yq = xqs[b], yqs[b]
        idx0, idx1 = idx0s[b], idx1s[b]
        wxl, wyl, msk = wxls[b], wyls[b], msks[b]

        @plsc.parallel_loop(0, NVEC, unroll=4)
        def _(v):
          off = v * VEC
          qx = xq[pl.ds(off, VEC)]
          qy = yq[pl.ds(off, VEC)]
          xlc, xw_l, _, mx = _search(cx, qx, W)
          ylc, yw_l, _, my = _search(cy, qy, H)
          p0 = ylc * W + xlc
          idx0[pl.ds(off, VEC)] = p0
          idx1[pl.ds(off, VEC)] = p0 + W
          wxl[pl.ds(off, VEC)] = xw_l
          wyl[pl.ds(off, VEC)] = yw_l
          msk[pl.ds(off, VEC)] = jnp.where(mx & my, 1.0, 0.0)

    def issue_g(i, b):
      @pl.when(valid(i))
      def _():
        pltpu.async_copy(rows8_hbm.at[idx0s[b]], g0s[b], semG[b])
        pltpu.async_copy(rows8_hbm.at[idx1s[b]], g1s[b], semG[b])

    def wait_g(i, b):
      @pl.when(valid(i))
      def _():
        pltpu.make_async_copy(rows8_hbm.at[idx0s[b]], g0s[b], semG[b]).wait()
        pltpu.make_async_copy(rows8_hbm.at[idx1s[b]], g1s[b], semG[b]).wait()

    def stage_reduce(i, b):
      @pl.when(valid(i))
      def _():
        g0, g1, ob = g0s[b], g1s[b], obs[b]
        wxl, wyl, msk = wxls[b], wyls[b], msks[b]

        @plsc.parallel_loop(0, NVEC, unroll=4)
        def _(v):
          off = v * VEC
          qidx = lanes + off
          axl = wxl[pl.ds(off, VEC)]
          ayl = wyl[pl.ds(off, VEC)]
          axu = 1.0 - axl
          ayu = 1.0 - ayl
          m = msk[pl.ds(off, VEC)]
          for c in range(C):
            cf = jnp.full((VEC,), c, jnp.int32)
            cf4 = jnp.full((VEC,), c + 4, jnp.int32)
            r00 = plsc.load_gather(g0, [qidx, cf])
            r01 = plsc.load_gather(g0, [qidx, cf4])
            r10 = plsc.load_gather(g1, [qidx, cf])
            r11 = plsc.load_gather(g1, [qidx, cf4])
            o = ayl * (axl * r00 + axu * r01) + ayu * (axl * r10 + axu * r11)
            o = jnp.where(m != 0.0, o, 0.0)
            ob[c, pl.ds(off, VEC)] = o

    def issue_e(i, b):
      @pl.when(valid(i))
      def _():
        base = qbase(i)
        for c in range(C):
          pltpu.async_copy(obs[b].at[c], out_hbms[c].at[pl.ds(base, CHUNK)],
                           semE[b])

    def wait_e(i, b):
      @pl.when(valid(i))
      def _():
        for c in range(C):
          pltpu.make_async_copy(
              obs[b].at[c], out_hbms[c].at[pl.ds(0, CHUNK)], semE[b]).wait()

    issue_a(0, 0)
    issue_a(1, 1)

    def pipe_body(i2, _):
      for b in range(2):
        i = i2 * 2 + b
        wait_a(i, b)
        stage_search(i, b)
        issue_g(i, b)
        issue_a(i + 2, b)
        j = i - 1
        bj = 1 - b
        wait_e(j - 2, bj)
        wait_g(j, bj)
        stage_reduce(j, bj)
        issue_e(j, bj)
      return 0

    lax.fori_loop(0, ITERS // 2, pipe_body, 0)

    last = ITERS - 1
    wait_e(last - 2, 1)
    wait_g(last, 1)
    stage_reduce(last, 1)
    issue_e(last, 1)
    wait_e(last - 1, 0)
    wait_e(last, 1)

  return kern


_interp = _make_kernel()

RBLK = 8192
RGRID = -(-N // RBLK)


def _retile_body(i0, i1, i2, i3, o):
  rows = [x[...].reshape(1, RBLK) for x in (i0, i1, i2, i3)]
  o[...] = jnp.concatenate(rows, axis=0)


def _retile(chans):
  return pl.pallas_call(
      _retile_body,
      out_shape=jax.ShapeDtypeStruct((C, N), jnp.float32),
      grid=(RGRID,),
      in_specs=[pl.BlockSpec((RBLK,), lambda j: (j,)) for _ in range(C)],
      out_specs=pl.BlockSpec((C, RBLK), lambda j: (0, j)),
  )(*chans)


@jax.jit
def kernel(x_coords, y_coords, x_query, y_query, z):
  rows8 = _prep(z.reshape(C * H * W))
  chans = _interp(x_coords, y_coords, x_query, y_query, rows8)
  return _retile(chans)


# speedup vs baseline: 1.5601x; 1.0602x over previous
import functools

import jax
import jax.numpy as jnp
from jax import lax
from jax.experimental import pallas as pl
from jax.experimental.pallas import tpu as pltpu
from jax.experimental.pallas import tpu_sc as plsc

W = 2048
H = 2048
C = 4
N = 2000000

NC = 2
NS = 16
NW = NC * NS
VEC = 16

CHUNK = 2000
NVEC = CHUNK // VEC
NCHUNKS = N // CHUNK
ITERS = (NCHUNKS + NW - 1) // NW


def _search(c_ref, q, n):
  lo = jnp.zeros((VEC,), jnp.int32)
  step = n >> 1
  while step:
    m = lo + step
    v = plsc.load_gather(c_ref, [m - 1])
    lo = jnp.where(v <= q, m, lo)
    step >>= 1
  cmax = plsc.load_gather(c_ref, [jnp.full((VEC,), n - 1, jnp.int32)])
  cnt = jnp.where(cmax <= q, n, lo)
  xl = cnt - 1
  valid = (xl >= 0) & (xl <= n - 2)
  xlc = jnp.clip(xl, 0, n - 2)
  cl = plsc.load_gather(c_ref, [xlc])
  cu = plsc.load_gather(c_ref, [xlc + 1])
  rd = 1.0 / (cu - cl)
  return xlc, (cu - q) * rd, (q - cl) * rd, valid


SEGP = 2048
PSEG = (H * W) // NW // SEGP


def _make_prep():
  mesh = plsc.VectorSubcoreMesh(core_axis_name="c", subcore_axis_name="s")

  @functools.partial(
      pl.kernel,
      out_type=jax.ShapeDtypeStruct((H * W, 8), jnp.float32),
      mesh=mesh,
      compiler_params=pltpu.CompilerParams(
          needs_layout_passes=False, use_tc_tiling_on_sc=False),
      scratch_types=[
          [[pltpu.VMEM((SEGP + 8,), jnp.float32) for _ in range(C)]
           for _ in range(2)],
          [pltpu.VMEM((SEGP, 8), jnp.float32) for _ in range(2)],
          [pltpu.SemaphoreType.DMA for _ in range(2)],
          [pltpu.SemaphoreType.DMA for _ in range(2)],
      ],
  )
  def prep(zl_hbm, rows_hbm, zbufs, obufs, semI, semO):
    wid = lax.axis_index("s") * NC + lax.axis_index("c")
    p_lo = wid * (PSEG * SEGP)
    lanes = lax.iota(jnp.int32, VEC)

    def issue_i(s, b):
      @pl.when(s < PSEG)
      def _():
        pbase = p_lo + s * SEGP
        for c in range(C):
          src = c * (H * W) + pbase
          if c == C - 1:
            is_edge = pbase == (H * W - SEGP)

            @pl.when(is_edge)
            def _():
              pltpu.async_copy(zl_hbm.at[pl.ds(src, SEGP)],
                               zbufs[b][c].at[pl.ds(0, SEGP)], semI[b])
              pltpu.async_copy(zl_hbm.at[pl.ds(0, 8)],
                               zbufs[b][c].at[pl.ds(SEGP, 8)], semI[b])

            @pl.when(jnp.logical_not(is_edge))
            def _():
              pltpu.async_copy(zl_hbm.at[pl.ds(src, SEGP + 8)], zbufs[b][c],
                               semI[b])
          else:
            pltpu.async_copy(zl_hbm.at[pl.ds(src, SEGP + 8)], zbufs[b][c],
                             semI[b])

    def wait_i(b):
      for c in range(C):
        pltpu.make_async_copy(zl_hbm.at[pl.ds(0, SEGP + 8)], zbufs[b][c],
                              semI[b]).wait()

    def wait_o(s, b):
      @pl.when(s >= 0)
      def _():
        pltpu.make_async_copy(obufs[b], rows_hbm.at[pl.ds(0, SEGP)],
                              semO[b]).wait()

    issue_i(0, 0)
    issue_i(1, 1)

    def seg_body(s2, _):
      for b in range(2):
        s = s2 * 2 + b
        wait_i(b)
        wait_o(s - 2, b)
        obuf = obufs[b]

        @plsc.parallel_loop(0, SEGP // VEC, unroll=4)
        def _(i):
          row = i * VEC + lanes
          for c in range(C):
            v0 = zbufs[b][c][pl.ds(i * VEC, VEC)]
            v1 = zbufs[b][c][pl.ds(i * VEC + 1, VEC)]
            plsc.store_scatter(obuf, [row, jnp.full((VEC,), c, jnp.int32)],
                               v0)
            plsc.store_scatter(obuf, [row, jnp.full((VEC,), c + 4, jnp.int32)],
                               v1)

        pbase = p_lo + s * SEGP
        pltpu.async_copy(obufs[b], rows_hbm.at[pl.ds(pbase, SEGP)], semO[b])
        issue_i(s + 2, b)
      return 0

    lax.fori_loop(0, PSEG // 2, seg_body, 0)
    wait_o(PSEG - 2, 0)
    wait_o(PSEG - 1, 1)

  return prep


_prep = _make_prep()


def _make_kernel():
  mesh = plsc.VectorSubcoreMesh(core_axis_name="c", subcore_axis_name="s")

  @functools.partial(
      pl.kernel,
      out_type=[jax.ShapeDtypeStruct((N,), jnp.float32) for _ in range(C)],
      mesh=mesh,
      compiler_params=pltpu.CompilerParams(
          needs_layout_passes=False, use_tc_tiling_on_sc=False),
      scratch_types=[
          pltpu.VMEM((W,), jnp.float32),
          pltpu.VMEM((H,), jnp.float32),
          [pltpu.VMEM((CHUNK,), jnp.float32) for _ in range(2)],
          [pltpu.VMEM((CHUNK,), jnp.float32) for _ in range(2)],
          [pltpu.VMEM((CHUNK,), jnp.int32) for _ in range(2)],
          [pltpu.VMEM((CHUNK,), jnp.int32) for _ in range(2)],
          [pltpu.VMEM((CHUNK,), jnp.float32) for _ in range(2)],
          [pltpu.VMEM((CHUNK,), jnp.float32) for _ in range(2)],
          [pltpu.VMEM((CHUNK,), jnp.float32) for _ in range(2)],
          [pltpu.VMEM((CHUNK, 8), jnp.float32) for _ in range(2)],
          [pltpu.VMEM((CHUNK, 8), jnp.float32) for _ in range(2)],
          [pltpu.VMEM((C, CHUNK), jnp.float32) for _ in range(2)],
          [pltpu.SemaphoreType.DMA for _ in range(2)],
          [pltpu.SemaphoreType.DMA for _ in range(2)],
          [pltpu.SemaphoreType.DMA for _ in range(2)],
      ],
  )
  def kern(cx_hbm, cy_hbm, xq_hbm, yq_hbm, rows8_hbm,
           o0_hbm, o1_hbm, o2_hbm, o3_hbm,
           cx, cy, xqs, yqs, idx0s, idx1s, wxls, wyls, msks, g0s, g1s, obs,
           semA, semG, semE):
    out_hbms = (o0_hbm, o1_hbm, o2_hbm, o3_hbm)
    wid = lax.axis_index("s") * NC + lax.axis_index("c")
    pltpu.sync_copy(cx_hbm, cx)
    pltpu.sync_copy(cy_hbm, cy)
    lanes = lax.iota(jnp.int32, VEC)

    def valid(i):
      return (jnp.asarray(i, jnp.int32) >= 0) & ((i * NW + wid) < NCHUNKS)

    def qbase(i):
      return (i * NW + wid) * CHUNK

    def issue_a(i, b):
      @pl.when(valid(i))
      def _():
        base = qbase(i)
        pltpu.async_copy(xq_hbm.at[pl.ds(base, CHUNK)], xqs[b], semA[b])
        pltpu.async_copy(yq_hbm.at[pl.ds(base, CHUNK)], yqs[b], semA[b])

    def wait_a(i, b):
      @pl.when(valid(i))
      def _():
        pltpu.make_async_copy(
            xq_hbm.at[pl.ds(0, CHUNK)], xqs[b], semA[b]).wait()
        pltpu.make_async_copy(
            yq_hbm.at[pl.ds(0, CHUNK)], yqs[b], semA[b]).wait()

    def stage_search(i, b):
      @pl.when(valid(i))
      def _():
        xq, yq = xqs[b], yqs[b]
        idx0, idx1 = idx0s[b], idx1s[b]
        wxl, wyl, msk = wxls[b], wyls[b], msks[b]

        @plsc.parallel_loop(0, NVEC, unroll=4)
        def _(v):
          off = v * VEC
          qx = xq[pl.ds(off, VEC)]
          qy = yq[pl.ds(off, VEC)]
          xlc, xw_l, _, mx = _search(cx, qx, W)
          ylc, yw_l, _, my = _search(cy, qy, H)
          p0 = ylc * W + xlc
          idx0[pl.ds(off, VEC)] = p0
          idx1[pl.ds(off, VEC)] = p0 + W
          wxl[pl.ds(off, VEC)] = xw_l
          wyl[pl.ds(off, VEC)] = yw_l
          msk[pl.ds(off, VEC)] = jnp.where(mx & my, 1.0, 0.0)

    def issue_g(i, b):
      @pl.when(valid(i))
      def _():
        pltpu.async_copy(rows8_hbm.at[idx0s[b]], g0s[b], semG[b])
        pltpu.async_copy(rows8_hbm.at[idx1s[b]], g1s[b], semG[b])

    def wait_g(i, b):
      @pl.when(valid(i))
      def _():
        pltpu.make_async_copy(rows8_hbm.at[idx0s[b]], g0s[b], semG[b]).wait()
        pltpu.make_async_copy(rows8_hbm.at[idx1s[b]], g1s[b], semG[b]).wait()

    def stage_reduce(i, b):
      @pl.when(valid(i))
      def _():
        g0, g1, ob = g0s[b], g1s[b], obs[b]
        wxl, wyl, msk = wxls[b], wyls[b], msks[b]

        @plsc.parallel_loop(0, NVEC, unroll=4)
        def _(v):
          off = v * VEC
          qidx = lanes + off
          axl = wxl[pl.ds(off, VEC)]
          ayl = wyl[pl.ds(off, VEC)]
          axu = 1.0 - axl
          ayu = 1.0 - ayl
          m = msk[pl.ds(off, VEC)]
          for c in range(C):
            cf = jnp.full((VEC,), c, jnp.int32)
            cf4 = jnp.full((VEC,), c + 4, jnp.int32)
            r00 = plsc.load_gather(g0, [qidx, cf])
            r01 = plsc.load_gather(g0, [qidx, cf4])
            r10 = plsc.load_gather(g1, [qidx, cf])
            r11 = plsc.load_gather(g1, [qidx, cf4])
            o = ayl * (axl * r00 + axu * r01) + ayu * (axl * r10 + axu * r11)
            o = jnp.where(m != 0.0, o, 0.0)
            ob[c, pl.ds(off, VEC)] = o

    def issue_e(i, b):
      @pl.when(valid(i))
      def _():
        base = qbase(i)
        for c in range(C):
          pltpu.async_copy(obs[b].at[c], out_hbms[c].at[pl.ds(base, CHUNK)],
                           semE[b])

    def wait_e(i, b):
      @pl.when(valid(i))
      def _():
        for c in range(C):
          pltpu.make_async_copy(
              obs[b].at[c], out_hbms[c].at[pl.ds(0, CHUNK)], semE[b]).wait()

    issue_a(0, 0)
    issue_a(1, 1)

    def pipe_body(i2, _):
      for b in range(2):
        i = i2 * 2 + b
        wait_a(i, b)
        stage_search(i, b)
        issue_g(i, b)
        issue_a(i + 2, b)
        j = i - 1
        bj = 1 - b
        wait_e(j - 2, bj)
        wait_g(j, bj)
        stage_reduce(j, bj)
        issue_e(j, bj)
      return 0

    lax.fori_loop(0, ITERS // 2, pipe_body, 0)

    last = ITERS - 1
    wait_e(last - 2, 1)
    wait_g(last, 1)
    stage_reduce(last, 1)
    issue_e(last, 1)
    wait_e(last - 1, 0)
    wait_e(last, 1)

  return kern


_interp = _make_kernel()

OCH = 3200
OCHUNKS = N // OCH
OITERS = (OCHUNKS + NW - 1) // NW


def _make_owrite():
  mesh = plsc.VectorSubcoreMesh(core_axis_name="c", subcore_axis_name="s")

  @functools.partial(
      pl.kernel,
      out_type=jax.ShapeDtypeStruct((C, N), jnp.float32),
      mesh=mesh,
      compiler_params=pltpu.CompilerParams(needs_layout_passes=False),
      scratch_types=[
          [pltpu.VMEM((OCH,), jnp.float32) for _ in range(C)],
          pltpu.VMEM((C, OCH), jnp.float32),
      ],
  )
  def owr(i0_hbm, i1_hbm, i2_hbm, i3_hbm, out_hbm, ibufs, obuf):
    in_hbms = (i0_hbm, i1_hbm, i2_hbm, i3_hbm)
    wid = lax.axis_index("s") * NC + lax.axis_index("c")

    def body(j, _):
      cid = j * NW + wid

      @pl.when(cid < OCHUNKS)
      def _():
        base = cid * OCH
        for c in range(C):
          pltpu.sync_copy(in_hbms[c].at[pl.ds(base, OCH)], ibufs[c])

        @plsc.parallel_loop(0, OCH // VEC, unroll=4)
        def _(k):
          off = k * VEC
          for c in range(C):
            obuf[c, pl.ds(off, VEC)] = ibufs[c][pl.ds(off, VEC)]

        pltpu.sync_copy(obuf, out_hbm.at[:, pl.ds(base, OCH)])
      return 0

    lax.fori_loop(0, OITERS, body, 0)

  return owr


_owrite = _make_owrite()


RBLK = 8192
RGRID = -(-N // RBLK)


def _retile_body(i0, i1, i2, i3, o):
  rows = [x[...].reshape(1, RBLK) for x in (i0, i1, i2, i3)]
  o[...] = jnp.concatenate(rows, axis=0)


def _retile(chans):
  return pl.pallas_call(
      _retile_body,
      out_shape=jax.ShapeDtypeStruct((C, N), jnp.float32),
      grid=(RGRID,),
      in_specs=[pl.BlockSpec((RBLK,), lambda j: (j,)) for _ in range(C)],
      out_specs=pl.BlockSpec((C, RBLK), lambda j: (0, j)),
  )(*chans)


@jax.jit
def kernel(x_coords, y_coords, x_query, y_query, z):
  rows8 = _prep(z.reshape(C * H * W))
  chans = _interp(x_coords, y_coords, x_query, y_query, rows8)
  return _owrite(*chans)


# speedup vs baseline: 2.0175x; 1.2932x over previous
import functools

import jax
import jax.numpy as jnp
from jax import lax
from jax.experimental import pallas as pl
from jax.experimental.pallas import tpu as pltpu
from jax.experimental.pallas import tpu_sc as plsc

W = 2048
H = 2048
C = 4
N = 2000000

NC = 2
NS = 16
NW = NC * NS
VEC = 16

CHUNK = 2000
NVEC = CHUNK // VEC
NCHUNKS = N // CHUNK
ITERS = (NCHUNKS + NW - 1) // NW


def _search(c_ref, cp, q, n):
  him = jnp.full((VEC,), -1, jnp.int32)
  for st in (8, 4, 2, 1):
    m1 = him + st
    v = jnp.take_along_axis(cp, m1, axis=0)
    him = jnp.where(v <= q, m1, him)
  lom = jnp.minimum(him, 14) * 128 + 127
  for st in (64, 32, 16, 8, 4, 2, 1):
    m1 = lom + st
    v = plsc.load_gather(c_ref, [m1])
    lom = jnp.where(v <= q, m1, lom)
  cmax = plsc.load_gather(c_ref, [jnp.full((VEC,), n - 1, jnp.int32)])
  xl = jnp.where(cmax <= q, n - 1, lom)
  valid = (xl >= 0) & (xl <= n - 2)
  xlc = jnp.clip(xl, 0, n - 2)
  cl = plsc.load_gather(c_ref, [xlc])
  cu = plsc.load_gather(c_ref, [xlc + 1])
  rd = 1.0 / (cu - cl)
  return xlc, (cu - q) * rd, valid


SEGP = 2048
PSEG = (H * W) // NW // SEGP


def _make_prep():
  mesh = plsc.VectorSubcoreMesh(core_axis_name="c", subcore_axis_name="s")

  @functools.partial(
      pl.kernel,
      out_type=jax.ShapeDtypeStruct((H * W, 8), jnp.float32),
      mesh=mesh,
      compiler_params=pltpu.CompilerParams(
          needs_layout_passes=False, use_tc_tiling_on_sc=False),
      scratch_types=[
          [[pltpu.VMEM((SEGP + 8,), jnp.float32) for _ in range(C)]
           for _ in range(2)],
          [pltpu.VMEM((SEGP, 8), jnp.float32) for _ in range(2)],
          [pltpu.SemaphoreType.DMA for _ in range(2)],
          [pltpu.SemaphoreType.DMA for _ in range(2)],
      ],
  )
  def prep(zl_hbm, rows_hbm, zbufs, obufs, semI, semO):
    wid = lax.axis_index("s") * NC + lax.axis_index("c")
    p_lo = wid * (PSEG * SEGP)
    lanes = lax.iota(jnp.int32, VEC)

    def issue_i(s, b):
      @pl.when(s < PSEG)
      def _():
        pbase = p_lo + s * SEGP
        for c in range(C):
          src = c * (H * W) + pbase
          if c == C - 1:
            is_edge = pbase == (H * W - SEGP)

            @pl.when(is_edge)
            def _():
              pltpu.async_copy(zl_hbm.at[pl.ds(src, SEGP)],
                               zbufs[b][c].at[pl.ds(0, SEGP)], semI[b])
              pltpu.async_copy(zl_hbm.at[pl.ds(0, 8)],
                               zbufs[b][c].at[pl.ds(SEGP, 8)], semI[b])

            @pl.when(jnp.logical_not(is_edge))
            def _():
              pltpu.async_copy(zl_hbm.at[pl.ds(src, SEGP + 8)], zbufs[b][c],
                               semI[b])
          else:
            pltpu.async_copy(zl_hbm.at[pl.ds(src, SEGP + 8)], zbufs[b][c],
                             semI[b])

    def wait_i(b):
      for c in range(C):
        pltpu.make_async_copy(zl_hbm.at[pl.ds(0, SEGP + 8)], zbufs[b][c],
                              semI[b]).wait()

    def wait_o(s, b):
      @pl.when(s >= 0)
      def _():
        pltpu.make_async_copy(obufs[b], rows_hbm.at[pl.ds(0, SEGP)],
                              semO[b]).wait()

    issue_i(0, 0)
    issue_i(1, 1)

    def seg_body(s2, _):
      for b in range(2):
        s = s2 * 2 + b
        wait_i(b)
        wait_o(s - 2, b)
        obuf = obufs[b]

        @plsc.parallel_loop(0, SEGP // VEC, unroll=4)
        def _(i):
          row = i * VEC + lanes
          for c in range(C):
            v0 = zbufs[b][c][pl.ds(i * VEC, VEC)]
            v1 = zbufs[b][c][pl.ds(i * VEC + 1, VEC)]
            plsc.store_scatter(obuf, [row, jnp.full((VEC,), c, jnp.int32)],
                               v0)
            plsc.store_scatter(obuf, [row, jnp.full((VEC,), c + 4, jnp.int32)],
                               v1)

        pbase = p_lo + s * SEGP
        pltpu.async_copy(obufs[b], rows_hbm.at[pl.ds(pbase, SEGP)], semO[b])
        issue_i(s + 2, b)
      return 0

    lax.fori_loop(0, PSEG // 2, seg_body, 0)
    wait_o(PSEG - 2, 0)
    wait_o(PSEG - 1, 1)

  return prep


_prep = _make_prep()


def _make_kernel():
  mesh = plsc.VectorSubcoreMesh(core_axis_name="c", subcore_axis_name="s")

  @functools.partial(
      pl.kernel,
      out_type=[jax.ShapeDtypeStruct((N,), jnp.float32) for _ in range(C)],
      mesh=mesh,
      compiler_params=pltpu.CompilerParams(
          needs_layout_passes=False, use_tc_tiling_on_sc=False),
      scratch_types=[
          pltpu.VMEM((W,), jnp.float32),
          pltpu.VMEM((H,), jnp.float32),
          [pltpu.VMEM((CHUNK,), jnp.float32) for _ in range(2)],
          [pltpu.VMEM((CHUNK,), jnp.float32) for _ in range(2)],
          [pltpu.VMEM((CHUNK,), jnp.int32) for _ in range(2)],
          [pltpu.VMEM((CHUNK,), jnp.int32) for _ in range(2)],
          [pltpu.VMEM((CHUNK,), jnp.float32) for _ in range(2)],
          [pltpu.VMEM((CHUNK,), jnp.float32) for _ in range(2)],
          [pltpu.VMEM((CHUNK,), jnp.float32) for _ in range(2)],
          [pltpu.VMEM((CHUNK, 8), jnp.float32) for _ in range(2)],
          [pltpu.VMEM((CHUNK, 8), jnp.float32) for _ in range(2)],
          [pltpu.VMEM((C, CHUNK), jnp.float32) for _ in range(2)],
          [pltpu.SemaphoreType.DMA for _ in range(2)],
          [pltpu.SemaphoreType.DMA for _ in range(2)],
          [pltpu.SemaphoreType.DMA for _ in range(2)],
      ],
  )
  def kern(cx_hbm, cy_hbm, xq_hbm, yq_hbm, rows8_hbm,
           o0_hbm, o1_hbm, o2_hbm, o3_hbm,
           cx, cy, xqs, yqs, idx0s, idx1s, wxls, wyls, msks, g0s, g1s, obs,
           semA, semG, semE):
    out_hbms = (o0_hbm, o1_hbm, o2_hbm, o3_hbm)
    wid = lax.axis_index("s") * NC + lax.axis_index("c")
    pltpu.sync_copy(cx_hbm, cx)
    pltpu.sync_copy(cy_hbm, cy)
    lanes = lax.iota(jnp.int32, VEC)
    cpx = plsc.load_gather(cx, [lanes * 128 + 127])
    cpy = plsc.load_gather(cy, [lanes * 128 + 127])

    def valid(i):
      return (jnp.asarray(i, jnp.int32) >= 0) & ((i * NW + wid) < NCHUNKS)

    def qbase(i):
      return (i * NW + wid) * CHUNK

    def issue_a(i, b):
      @pl.when(valid(i))
      def _():
        base = qbase(i)
        pltpu.async_copy(xq_hbm.at[pl.ds(base, CHUNK)], xqs[b], semA[b])
        pltpu.async_copy(yq_hbm.at[pl.ds(base, CHUNK)], yqs[b], semA[b])

    def wait_a(i, b):
      @pl.when(valid(i))
      def _():
        pltpu.make_async_copy(
            xq_hbm.at[pl.ds(0, CHUNK)], xqs[b], semA[b]).wait()
        pltpu.make_async_copy(
            yq_hbm.at[pl.ds(0, CHUNK)], yqs[b], semA[b]).wait()

    def stage_search(i, b):
      @pl.when(valid(i))
      def _():
        xq, yq = xqs[b], yqs[b]
        idx0, idx1 = idx0s[b], idx1s[b]
        wxl, wyl, msk = wxls[b], wyls[b], msks[b]

        @plsc.parallel_loop(0, NVEC, unroll=4)
        def _(v):
          off = v * VEC
          qx = xq[pl.ds(off, VEC)]
          qy = yq[pl.ds(off, VEC)]
          xlc, xw_l, mx = _search(cx, cpx, qx, W)
          ylc, yw_l, my = _search(cy, cpy, qy, H)
          p0 = ylc * W + xlc
          idx0[pl.ds(off, VEC)] = p0
          idx1[pl.ds(off, VEC)] = p0 + W
          wxl[pl.ds(off, VEC)] = xw_l
          wyl[pl.ds(off, VEC)] = yw_l
          msk[pl.ds(off, VEC)] = jnp.where(mx & my, 1.0, 0.0)

    def issue_g(i, b):
      @pl.when(valid(i))
      def _():
        pltpu.async_copy(rows8_hbm.at[idx0s[b]], g0s[b], semG[b])
        pltpu.async_copy(rows8_hbm.at[idx1s[b]], g1s[b], semG[b])

    def wait_g(i, b):
      @pl.when(valid(i))
      def _():
        pltpu.make_async_copy(rows8_hbm.at[idx0s[b]], g0s[b], semG[b]).wait()
        pltpu.make_async_copy(rows8_hbm.at[idx1s[b]], g1s[b], semG[b]).wait()

    def stage_reduce(i, b):
      @pl.when(valid(i))
      def _():
        g0, g1, ob = g0s[b], g1s[b], obs[b]
        wxl, wyl, msk = wxls[b], wyls[b], msks[b]

        @plsc.parallel_loop(0, NVEC, unroll=4)
        def _(v):
          off = v * VEC
          qidx = lanes + off
          axl = wxl[pl.ds(off, VEC)]
          ayl = wyl[pl.ds(off, VEC)]
          axu = 1.0 - axl
          ayu = 1.0 - ayl
          m = msk[pl.ds(off, VEC)]
          for c in range(C):
            cf = jnp.full((VEC,), c, jnp.int32)
            cf4 = jnp.full((VEC,), c + 4, jnp.int32)
            r00 = plsc.load_gather(g0, [qidx, cf])
            r01 = plsc.load_gather(g0, [qidx, cf4])
            r10 = plsc.load_gather(g1, [qidx, cf])
            r11 = plsc.load_gather(g1, [qidx, cf4])
            o = ayl * (axl * r00 + axu * r01) + ayu * (axl * r10 + axu * r11)
            o = jnp.where(m != 0.0, o, 0.0)
            ob[c, pl.ds(off, VEC)] = o

    def issue_e(i, b):
      @pl.when(valid(i))
      def _():
        base = qbase(i)
        for c in range(C):
          pltpu.async_copy(obs[b].at[c], out_hbms[c].at[pl.ds(base, CHUNK)],
                           semE[b])

    def wait_e(i, b):
      @pl.when(valid(i))
      def _():
        for c in range(C):
          pltpu.make_async_copy(
              obs[b].at[c], out_hbms[c].at[pl.ds(0, CHUNK)], semE[b]).wait()

    issue_a(0, 0)
    issue_a(1, 1)

    def pipe_body(i2, _):
      for b in range(2):
        i = i2 * 2 + b
        wait_a(i, b)
        stage_search(i, b)
        issue_g(i, b)
        issue_a(i + 2, b)
        j = i - 1
        bj = 1 - b
        wait_e(j - 2, bj)
        wait_g(j, bj)
        stage_reduce(j, bj)
        issue_e(j, bj)
      return 0

    lax.fori_loop(0, ITERS // 2, pipe_body, 0)

    last = ITERS - 1
    wait_e(last - 2, 1)
    wait_g(last, 1)
    stage_reduce(last, 1)
    issue_e(last, 1)
    wait_e(last - 1, 0)
    wait_e(last, 1)

  return kern


_interp = _make_kernel()

OCH = 3200
OCHUNKS = N // OCH
OITERS = (OCHUNKS + NW - 1) // NW


def _make_owrite():
  mesh = plsc.VectorSubcoreMesh(core_axis_name="c", subcore_axis_name="s")

  @functools.partial(
      pl.kernel,
      out_type=jax.ShapeDtypeStruct((C, N), jnp.float32),
      mesh=mesh,
      compiler_params=pltpu.CompilerParams(needs_layout_passes=False),
      scratch_types=[
          [pltpu.VMEM((OCH,), jnp.float32) for _ in range(C)],
          pltpu.VMEM((C, OCH), jnp.float32),
      ],
  )
  def owr(i0_hbm, i1_hbm, i2_hbm, i3_hbm, out_hbm, ibufs, obuf):
    in_hbms = (i0_hbm, i1_hbm, i2_hbm, i3_hbm)
    wid = lax.axis_index("s") * NC + lax.axis_index("c")

    def body(j, _):
      cid = j * NW + wid

      @pl.when(cid < OCHUNKS)
      def _():
        base = cid * OCH
        for c in range(C):
          pltpu.sync_copy(in_hbms[c].at[pl.ds(base, OCH)], ibufs[c])

        @plsc.parallel_loop(0, OCH // VEC, unroll=4)
        def _(k):
          off = k * VEC
          for c in range(C):
            obuf[c, pl.ds(off, VEC)] = ibufs[c][pl.ds(off, VEC)]

        pltpu.sync_copy(obuf, out_hbm.at[:, pl.ds(base, OCH)])
      return 0

    lax.fori_loop(0, OITERS, body, 0)

  return owr


_owrite = _make_owrite()


RBLK = 8192
RGRID = -(-N // RBLK)


def _retile_body(i0, i1, i2, i3, o):
  rows = [x[...].reshape(1, RBLK) for x in (i0, i1, i2, i3)]
  o[...] = jnp.concatenate(rows, axis=0)


def _retile(chans):
  return pl.pallas_call(
      _retile_body,
      out_shape=jax.ShapeDtypeStruct((C, N), jnp.float32),
      grid=(RGRID,),
      in_specs=[pl.BlockSpec((RBLK,), lambda j: (j,)) for _ in range(C)],
      out_specs=pl.BlockSpec((C, RBLK), lambda j: (0, j)),
  )(*chans)


@jax.jit
def kernel(x_coords, y_coords, x_query, y_query, z):
  rows8 = _prep(z.reshape(C * H * W))
  chans = _interp(x_coords, y_coords, x_query, y_query, rows8)
  return _owrite(*chans)


# speedup vs baseline: 2.2042x; 1.0925x over previous
import functools

import jax
import jax.numpy as jnp
from jax import lax
from jax.experimental import pallas as pl
from jax.experimental.pallas import tpu as pltpu
from jax.experimental.pallas import tpu_sc as plsc

W = 2048
H = 2048
C = 4
N = 2000000

NC = 2
NS = 16
NW = NC * NS
VEC = 16

CHUNK = 2000
NVEC = CHUNK // VEC
NCHUNKS = N // CHUNK
ITERS = (NCHUNKS + NW - 1) // NW


def _search(c_ref, cp, q, n):
  him = jnp.full((VEC,), -1, jnp.int32)
  for st in (8, 4, 2, 1):
    m1 = him + st
    v = jnp.take_along_axis(cp, m1, axis=0)
    him = jnp.where(v <= q, m1, him)
  lom = jnp.minimum(him, 14) * 128 + 127
  for st in (64, 32, 16, 8, 4, 2, 1):
    m1 = lom + st
    v = plsc.load_gather(c_ref, [m1])
    lom = jnp.where(v <= q, m1, lom)
  cmax = plsc.load_gather(c_ref, [jnp.full((VEC,), n - 1, jnp.int32)])
  xl = jnp.where(cmax <= q, n - 1, lom)
  valid = (xl >= 0) & (xl <= n - 2)
  xlc = jnp.clip(xl, 0, n - 2)
  cl = plsc.load_gather(c_ref, [xlc])
  cu = plsc.load_gather(c_ref, [xlc + 1])
  rd = 1.0 / (cu - cl)
  return xlc, (cu - q) * rd, valid


SEGP = 2048
PSEG = (H * W) // NW // SEGP


def _make_prep():
  mesh = plsc.VectorSubcoreMesh(core_axis_name="c", subcore_axis_name="s")

  @functools.partial(
      pl.kernel,
      out_type=jax.ShapeDtypeStruct((H * W, 8), jnp.float32),
      mesh=mesh,
      compiler_params=pltpu.CompilerParams(
          needs_layout_passes=False, use_tc_tiling_on_sc=False),
      scratch_types=[
          [[pltpu.VMEM((SEGP + 8,), jnp.float32) for _ in range(C)]
           for _ in range(2)],
          [pltpu.VMEM((SEGP, 8), jnp.float32) for _ in range(2)],
          [pltpu.SemaphoreType.DMA for _ in range(2)],
          [pltpu.SemaphoreType.DMA for _ in range(2)],
      ],
  )
  def prep(zl_hbm, rows_hbm, zbufs, obufs, semI, semO):
    wid = lax.axis_index("s") * NC + lax.axis_index("c")
    p_lo = wid * (PSEG * SEGP)
    lanes = lax.iota(jnp.int32, VEC)

    def issue_i(s, b):
      @pl.when(s < PSEG)
      def _():
        pbase = p_lo + s * SEGP
        for c in range(C):
          src = c * (H * W) + pbase
          if c == C - 1:
            is_edge = pbase == (H * W - SEGP)

            @pl.when(is_edge)
            def _():
              pltpu.async_copy(zl_hbm.at[pl.ds(src, SEGP)],
                               zbufs[b][c].at[pl.ds(0, SEGP)], semI[b])
              pltpu.async_copy(zl_hbm.at[pl.ds(0, 8)],
                               zbufs[b][c].at[pl.ds(SEGP, 8)], semI[b])

            @pl.when(jnp.logical_not(is_edge))
            def _():
              pltpu.async_copy(zl_hbm.at[pl.ds(src, SEGP + 8)], zbufs[b][c],
                               semI[b])
          else:
            pltpu.async_copy(zl_hbm.at[pl.ds(src, SEGP + 8)], zbufs[b][c],
                             semI[b])

    def wait_i(b):
      for c in range(C):
        pltpu.make_async_copy(zl_hbm.at[pl.ds(0, SEGP + 8)], zbufs[b][c],
                              semI[b]).wait()

    def wait_o(s, b):
      @pl.when(s >= 0)
      def _():
        pltpu.make_async_copy(obufs[b], rows_hbm.at[pl.ds(0, SEGP)],
                              semO[b]).wait()

    issue_i(0, 0)
    issue_i(1, 1)

    def seg_body(s2, _):
      for b in range(2):
        s = s2 * 2 + b
        wait_i(b)
        wait_o(s - 2, b)
        obuf = obufs[b]

        @plsc.parallel_loop(0, SEGP // VEC, unroll=4)
        def _(i):
          row = i * VEC + lanes
          for c in range(C):
            v0 = zbufs[b][c][pl.ds(i * VEC, VEC)]
            v1 = zbufs[b][c][pl.ds(i * VEC + 1, VEC)]
            plsc.store_scatter(obuf, [row, jnp.full((VEC,), c, jnp.int32)],
                               v0)
            plsc.store_scatter(obuf, [row, jnp.full((VEC,), c + 4, jnp.int32)],
                               v1)

        pbase = p_lo + s * SEGP
        pltpu.async_copy(obufs[b], rows_hbm.at[pl.ds(pbase, SEGP)], semO[b])
        issue_i(s + 2, b)
      return 0

    lax.fori_loop(0, PSEG // 2, seg_body, 0)
    wait_o(PSEG - 2, 0)
    wait_o(PSEG - 1, 1)

  return prep


_prep = _make_prep()


def _make_kernel():
  mesh = plsc.VectorSubcoreMesh(core_axis_name="c", subcore_axis_name="s")

  @functools.partial(
      pl.kernel,
      out_type=[jax.ShapeDtypeStruct((N,), jnp.float32) for _ in range(C)],
      mesh=mesh,
      compiler_params=pltpu.CompilerParams(
          needs_layout_passes=False, use_tc_tiling_on_sc=False),
      scratch_types=[
          pltpu.VMEM((W,), jnp.float32),
          pltpu.VMEM((H,), jnp.float32),
          [pltpu.VMEM((CHUNK,), jnp.float32) for _ in range(2)],
          [pltpu.VMEM((CHUNK,), jnp.float32) for _ in range(2)],
          [pltpu.VMEM((CHUNK,), jnp.int32) for _ in range(2)],
          [pltpu.VMEM((CHUNK,), jnp.int32) for _ in range(2)],
          [pltpu.VMEM((CHUNK,), jnp.float32) for _ in range(2)],
          [pltpu.VMEM((CHUNK,), jnp.float32) for _ in range(2)],
          [pltpu.VMEM((CHUNK,), jnp.float32) for _ in range(2)],
          [pltpu.VMEM((CHUNK, 8), jnp.float32) for _ in range(2)],
          [pltpu.VMEM((CHUNK, 8), jnp.float32) for _ in range(2)],
          [pltpu.VMEM((C, CHUNK), jnp.float32) for _ in range(2)],
          [pltpu.SemaphoreType.DMA for _ in range(2)],
          [pltpu.SemaphoreType.DMA for _ in range(2)],
          [pltpu.SemaphoreType.DMA for _ in range(2)],
      ],
  )
  def kern(cx_hbm, cy_hbm, xq_hbm, yq_hbm, rows8_hbm,
           o0_hbm, o1_hbm, o2_hbm, o3_hbm,
           cx, cy, xqs, yqs, idx0s, idx1s, wxls, wyls, msks, g0s, g1s, obs,
           semA, semG, semE):
    out_hbms = (o0_hbm, o1_hbm, o2_hbm, o3_hbm)
    wid = lax.axis_index("s") * NC + lax.axis_index("c")
    pltpu.sync_copy(cx_hbm, cx)
    pltpu.sync_copy(cy_hbm, cy)
    lanes = lax.iota(jnp.int32, VEC)
    cpx = plsc.load_gather(cx, [lanes * 128 + 127])
    cpy = plsc.load_gather(cy, [lanes * 128 + 127])

    def valid(i):
      return (jnp.asarray(i, jnp.int32) >= 0) & ((i * NW + wid) < NCHUNKS)

    def qbase(i):
      return (i * NW + wid) * CHUNK

    def issue_a(i, b):
      @pl.when(valid(i))
      def _():
        base = qbase(i)
        pltpu.async_copy(xq_hbm.at[pl.ds(base, CHUNK)], xqs[b], semA[b])
        pltpu.async_copy(yq_hbm.at[pl.ds(base, CHUNK)], yqs[b], semA[b])

    def wait_a(i, b):
      @pl.when(valid(i))
      def _():
        pltpu.make_async_copy(
            xq_hbm.at[pl.ds(0, CHUNK)], xqs[b], semA[b]).wait()
        pltpu.make_async_copy(
            yq_hbm.at[pl.ds(0, CHUNK)], yqs[b], semA[b]).wait()

    def stage_search(i, b):
      @pl.when(valid(i))
      def _():
        xq, yq = xqs[b], yqs[b]
        idx0, idx1 = idx0s[b], idx1s[b]
        wxl, wyl, msk = wxls[b], wyls[b], msks[b]

        @plsc.parallel_loop(0, NVEC, unroll=4)
        def _(v):
          off = v * VEC
          qx = xq[pl.ds(off, VEC)]
          qy = yq[pl.ds(off, VEC)]
          xlc, xw_l, mx = _search(cx, cpx, qx, W)
          ylc, yw_l, my = _search(cy, cpy, qy, H)
          p0 = ylc * W + xlc
          idx0[pl.ds(off, VEC)] = p0
          idx1[pl.ds(off, VEC)] = p0 + W
          wxl[pl.ds(off, VEC)] = xw_l
          wyl[pl.ds(off, VEC)] = yw_l
          msk[pl.ds(off, VEC)] = jnp.where(mx & my, 1.0, 0.0)

    def issue_g(i, b):
      @pl.when(valid(i))
      def _():
        pltpu.async_copy(rows8_hbm.at[idx0s[b]], g0s[b], semG[b])
        pltpu.async_copy(rows8_hbm.at[idx1s[b]], g1s[b], semG[b])

    def wait_g(i, b):
      @pl.when(valid(i))
      def _():
        pltpu.make_async_copy(rows8_hbm.at[idx0s[b]], g0s[b], semG[b]).wait()
        pltpu.make_async_copy(rows8_hbm.at[idx1s[b]], g1s[b], semG[b]).wait()

    def stage_reduce(i, b):
      @pl.when(valid(i))
      def _():
        g0, g1, ob = g0s[b], g1s[b], obs[b]
        wxl, wyl, msk = wxls[b], wyls[b], msks[b]

        @plsc.parallel_loop(0, NVEC, unroll=4)
        def _(v):
          off = v * VEC
          qidx = lanes + off
          axl = wxl[pl.ds(off, VEC)]
          ayl = wyl[pl.ds(off, VEC)]
          axu = 1.0 - axl
          ayu = 1.0 - ayl
          m = msk[pl.ds(off, VEC)]
          for c in range(C):
            cf = jnp.full((VEC,), c, jnp.int32)
            cf4 = jnp.full((VEC,), c + 4, jnp.int32)
            r00 = plsc.load_gather(g0, [qidx, cf])
            r01 = plsc.load_gather(g0, [qidx, cf4])
            r10 = plsc.load_gather(g1, [qidx, cf])
            r11 = plsc.load_gather(g1, [qidx, cf4])
            o = ayl * (axl * r00 + axu * r01) + ayu * (axl * r10 + axu * r11)
            o = jnp.where(m != 0.0, o, 0.0)
            ob[c, pl.ds(off, VEC)] = o

    def issue_e(i, b):
      @pl.when(valid(i))
      def _():
        base = qbase(i)
        for c in range(C):
          pltpu.async_copy(obs[b].at[c], out_hbms[c].at[pl.ds(base, CHUNK)],
                           semE[b])

    def wait_e(i, b):
      @pl.when(valid(i))
      def _():
        for c in range(C):
          pltpu.make_async_copy(
              obs[b].at[c], out_hbms[c].at[pl.ds(0, CHUNK)], semE[b]).wait()

    issue_a(0, 0)
    issue_a(1, 1)

    def pipe_body(i2, _):
      for b in range(2):
        i = i2 * 2 + b
        wait_a(i, b)
        stage_search(i, b)
        issue_g(i, b)
        issue_a(i + 2, b)
        j = i - 1
        bj = 1 - b
        wait_e(j - 2, bj)
        wait_g(j, bj)
        stage_reduce(j, bj)
        issue_e(j, bj)
      return 0

    lax.fori_loop(0, ITERS // 2, pipe_body, 0)

    last = ITERS - 1
    wait_e(last - 2, 1)
    wait_g(last, 1)
    stage_reduce(last, 1)
    issue_e(last, 1)
    wait_e(last - 1, 0)
    wait_e(last, 1)

  return kern


_interp = _make_kernel()

OCH = 3200
OCHUNKS = N // OCH
OITERS = (OCHUNKS + NW - 1) // NW


def _make_owrite():
  mesh = plsc.VectorSubcoreMesh(core_axis_name="c", subcore_axis_name="s")

  @functools.partial(
      pl.kernel,
      out_type=jax.ShapeDtypeStruct((C, N), jnp.float32),
      mesh=mesh,
      compiler_params=pltpu.CompilerParams(needs_layout_passes=False),
      scratch_types=[
          [[pltpu.VMEM((OCH,), jnp.float32) for _ in range(C)]
           for _ in range(2)],
          [pltpu.VMEM((C, OCH), jnp.float32) for _ in range(2)],
          [pltpu.SemaphoreType.DMA for _ in range(2)],
          [pltpu.SemaphoreType.DMA for _ in range(2)],
      ],
  )
  def owr(i0_hbm, i1_hbm, i2_hbm, i3_hbm, out_hbm, ibufs, obufs, semI, semO):
    in_hbms = (i0_hbm, i1_hbm, i2_hbm, i3_hbm)
    wid = lax.axis_index("s") * NC + lax.axis_index("c")

    def valid(j):
      return (jnp.asarray(j, jnp.int32) >= 0) & ((j * NW + wid) < OCHUNKS)

    def issue_i(j, b):
      @pl.when(valid(j))
      def _():
        base = (j * NW + wid) * OCH
        for c in range(C):
          pltpu.async_copy(in_hbms[c].at[pl.ds(base, OCH)], ibufs[b][c],
                           semI[b])

    def wait_i(j, b):
      @pl.when(valid(j))
      def _():
        for c in range(C):
          pltpu.make_async_copy(in_hbms[c].at[pl.ds(0, OCH)], ibufs[b][c],
                                semI[b]).wait()

    def wait_o(j, b):
      @pl.when(valid(j))
      def _():
        pltpu.make_async_copy(obufs[b], out_hbm.at[:, pl.ds(0, OCH)],
                              semO[b]).wait()

    issue_i(0, 0)
    issue_i(1, 1)

    def body(j2, _):
      for b in range(2):
        j = j2 * 2 + b
        wait_i(j, b)
        wait_o(j - 2, b)

        @pl.when(valid(j))
        def _():
          obuf = obufs[b]

          @plsc.parallel_loop(0, OCH // VEC, unroll=4)
          def _(k):
            off = k * VEC
            for c in range(C):
              obuf[c, pl.ds(off, VEC)] = ibufs[b][c][pl.ds(off, VEC)]

          base = (j * NW + wid) * OCH
          pltpu.async_copy(obufs[b], out_hbm.at[:, pl.ds(base, OCH)], semO[b])
        issue_i(j + 2, b)
      return 0

    lax.fori_loop(0, OITERS // 2, body, 0)
    wait_o(OITERS - 2, 0)
    wait_o(OITERS - 1, 1)

  return owr


_owrite = _make_owrite()


RBLK = 8192
RGRID = -(-N // RBLK)


def _retile_body(i0, i1, i2, i3, o):
  rows = [x[...].reshape(1, RBLK) for x in (i0, i1, i2, i3)]
  o[...] = jnp.concatenate(rows, axis=0)


def _retile(chans):
  return pl.pallas_call(
      _retile_body,
      out_shape=jax.ShapeDtypeStruct((C, N), jnp.float32),
      grid=(RGRID,),
      in_specs=[pl.BlockSpec((RBLK,), lambda j: (j,)) for _ in range(C)],
      out_specs=pl.BlockSpec((C, RBLK), lambda j: (0, j)),
  )(*chans)


@jax.jit
def kernel(x_coords, y_coords, x_query, y_query, z):
  rows8 = _prep(z.reshape(C * H * W))
  chans = _interp(x_coords, y_coords, x_query, y_query, rows8)
  return _owrite(*chans)
